# baseline (speedup 1.0000x reference)
import functools

import jax
import jax.numpy as jnp
from jax import lax
from jax.experimental import pallas as pl
from jax.experimental.pallas import tpu as pltpu
from jax.experimental.pallas import tpu_sc as plsc

D_MODEL = 1024
NUM_CORES = 2
NUM_SUBCORES = 16
NUM_WORKERS = NUM_CORES * NUM_SUBCORES


@functools.lru_cache(maxsize=None)
def _make_gather(B: int, C: int, NBUF: int, B_out: int | None = None):
    if B_out is None:
        B_out = B
    b_per_w = B // NUM_WORKERS
    n_chunks = b_per_w // C
    mesh = plsc.VectorSubcoreMesh(
        core_axis_name="c",
        subcore_axis_name="s",
        num_cores=NUM_CORES,
        num_subcores=NUM_SUBCORES,
    )

    @functools.partial(
        pl.kernel,
        out_type=jax.ShapeDtypeStruct((B_out, D_MODEL), jnp.float32),
        mesh=mesh,
        scratch_types=[
            pltpu.VMEM((b_per_w,), jnp.int32),
            pltpu.VMEM((NBUF, C, D_MODEL), jnp.float32),
            [pltpu.SemaphoreType.DMA] * NBUF,
            [pltpu.SemaphoreType.DMA] * NBUF,
        ],
    )
    def gather_kernel(table_hbm, idx_hbm, out_hbm, idx_v, rows, gsems, wsems):
        wid = lax.axis_index("s") * NUM_CORES + lax.axis_index("c")
        base = wid * b_per_w
        pltpu.sync_copy(idx_hbm.at[pl.ds(base, b_per_w)], idx_v)

        def start_gather(c):
            b = c % NBUF
            return pltpu.async_copy(
                table_hbm.at[idx_v.at[pl.ds(c * C, C)]], rows.at[b], gsems[b]
            )

        gops = [None] * n_chunks
        wops = [None] * n_chunks
        for c in range(min(NBUF, n_chunks)):
            gops[c] = start_gather(c)
        for c in range(n_chunks):
            b = c % NBUF
            gops[c].wait()
            wops[c] = pltpu.async_copy(
                rows.at[b], out_hbm.at[pl.ds(base + c * C, C)], wsems[b]
            )
            if c + NBUF < n_chunks:
                wops[c].wait()
                gops[c + NBUF] = start_gather(c + NBUF)
        for c in range(max(0, n_chunks - NBUF), n_chunks):
            wops[c].wait()

    return gather_kernel


import math
import numpy as np


def _sinusoid_consts():
    div_term = np.exp(
        np.arange(0.0, D_MODEL, 2, dtype=np.float32)
        * (-math.log(10000.0) / D_MODEL)
    )
    dtf = np.repeat(div_term, 2).astype(np.float32)
    ph = np.tile(np.array([0.0, np.pi / 2], dtype=np.float32), D_MODEL // 2)
    return jnp.asarray(dtf[None, :]), jnp.asarray(ph[None, :])


def _make_sin_kernel(B: int, R: int):
    grid = B // R

    def body(pos_ref, dtf_ref, ph_ref, out_ref):
        out_ref[...] = jnp.sin(pos_ref[...] * dtf_ref[...] + ph_ref[...])

    return pl.pallas_call(
        body,
        grid=(grid,),
        in_specs=[
            pl.BlockSpec((R, 1), lambda i: (i, 0)),
            pl.BlockSpec((1, D_MODEL), lambda i: (0, 0)),
            pl.BlockSpec((1, D_MODEL), lambda i: (0, 0)),
        ],
        out_specs=pl.BlockSpec((R, D_MODEL), lambda i: (i, 0)),
        out_shape=jax.ShapeDtypeStruct((B, D_MODEL), jnp.float32),
    )


def kernel(x, p2e):
    shp = x.shape
    pos = x.reshape(-1, 1).astype(jnp.float32)
    B = pos.shape[0]
    dtf, ph = _sinusoid_consts()
    out = _make_sin_kernel(B, 256)(pos, dtf, ph)
    return out.reshape(shp + (D_MODEL,))


# speedup vs baseline: 4.0241x; 4.0241x over previous
import functools

import jax
import jax.numpy as jnp
from jax import lax
from jax.experimental import pallas as pl
from jax.experimental.pallas import tpu as pltpu
from jax.experimental.pallas import tpu_sc as plsc

D_MODEL = 1024
NUM_CORES = 2
NUM_SUBCORES = 16
NUM_WORKERS = NUM_CORES * NUM_SUBCORES


@functools.lru_cache(maxsize=None)
def _make_gather(B: int, C: int, NBUF: int):
    b_per_w = B // NUM_WORKERS
    n_chunks = b_per_w // C
    mesh = plsc.VectorSubcoreMesh(
        core_axis_name="c",
        subcore_axis_name="s",
        num_cores=NUM_CORES,
        num_subcores=NUM_SUBCORES,
    )

    @functools.partial(
        pl.kernel,
        out_type=jax.ShapeDtypeStruct((B, D_MODEL), jnp.float32),
        mesh=mesh,
        scratch_types=[
            pltpu.VMEM((b_per_w,), jnp.int32),
            pltpu.VMEM((NBUF, C, D_MODEL), jnp.float32),
            [pltpu.SemaphoreType.DMA] * NBUF,
            [pltpu.SemaphoreType.DMA] * NBUF,
        ],
    )
    def gather_kernel(table_hbm, idx_hbm, out_hbm, idx_v, rows, gsems, wsems):
        wid = lax.axis_index("s") * NUM_CORES + lax.axis_index("c")
        base = wid * b_per_w
        pltpu.sync_copy(idx_hbm.at[pl.ds(base, b_per_w)], idx_v)

        def start_gather(c):
            b = c % NBUF
            return pltpu.async_copy(
                table_hbm.at[idx_v.at[pl.ds(c * C, C)]], rows.at[b], gsems[b]
            )

        gops = [None] * n_chunks
        wops = [None] * n_chunks
        for c in range(min(NBUF, n_chunks)):
            gops[c] = start_gather(c)
        for c in range(n_chunks):
            b = c % NBUF
            gops[c].wait()
            wops[c] = pltpu.async_copy(
                rows.at[b], out_hbm.at[pl.ds(base + c * C, C)], wsems[b]
            )
            if c + NBUF < n_chunks:
                wops[c].wait()
                gops[c + NBUF] = start_gather(c + NBUF)
        for c in range(max(0, n_chunks - NBUF), n_chunks):
            wops[c].wait()

    return gather_kernel


def kernel(x, p2e):
    shp = x.shape
    idx = x.reshape(-1).astype(jnp.int32)
    out = _make_gather(idx.shape[0], 16, 6)(p2e, idx)
    return out.reshape(shp + (D_MODEL,))


# speedup vs baseline: 4.0301x; 1.0015x over previous
import functools

import jax
import jax.numpy as jnp
from jax import lax
from jax.experimental import pallas as pl
from jax.experimental.pallas import tpu as pltpu
from jax.experimental.pallas import tpu_sc as plsc

D_MODEL = 1024
NUM_CORES = 2
NUM_SUBCORES = 16
NUM_WORKERS = NUM_CORES * NUM_SUBCORES


@functools.lru_cache(maxsize=None)
def _make_gather(B: int, C: int, NBUF: int):
    b_per_w = B // NUM_WORKERS
    chunks = []
    off = 0
    while off < b_per_w:
        sz = min(C, b_per_w - off)
        chunks.append((off, sz))
        off += sz
    n_chunks = len(chunks)
    mesh = plsc.VectorSubcoreMesh(
        core_axis_name="c",
        subcore_axis_name="s",
        num_cores=NUM_CORES,
        num_subcores=NUM_SUBCORES,
    )

    @functools.partial(
        pl.kernel,
        out_type=jax.ShapeDtypeStruct((B, D_MODEL), jnp.float32),
        mesh=mesh,
        scratch_types=[
            pltpu.VMEM((b_per_w,), jnp.int32),
            pltpu.VMEM((NBUF, C, D_MODEL), jnp.float32),
            [pltpu.SemaphoreType.DMA] * NBUF,
            [pltpu.SemaphoreType.DMA] * NBUF,
        ],
    )
    def gather_kernel(table_hbm, idx_hbm, out_hbm, idx_v, rows, gsems, wsems):
        wid = lax.axis_index("s") * NUM_CORES + lax.axis_index("c")
        base = wid * b_per_w
        pltpu.sync_copy(idx_hbm.at[pl.ds(base, b_per_w)], idx_v)

        def start_gather(c):
            b = c % NBUF
            coff, csz = chunks[c]
            return pltpu.async_copy(
                table_hbm.at[idx_v.at[pl.ds(coff, csz)]],
                rows.at[b, pl.ds(0, csz)],
                gsems[b],
            )

        gops = [None] * n_chunks
        wops = [None] * n_chunks
        for c in range(min(NBUF, n_chunks)):
            gops[c] = start_gather(c)
        for c in range(n_chunks):
            b = c % NBUF
            coff, csz = chunks[c]
            gops[c].wait()
            wops[c] = pltpu.async_copy(
                rows.at[b, pl.ds(0, csz)],
                out_hbm.at[pl.ds(base + coff, csz)],
                wsems[b],
            )
            if c + NBUF < n_chunks:
                wops[c].wait()
                gops[c + NBUF] = start_gather(c + NBUF)
        for c in range(max(0, n_chunks - NBUF), n_chunks):
            wops[c].wait()

    return gather_kernel


def kernel(x, p2e):
    shp = x.shape
    idx = x.reshape(-1).astype(jnp.int32)
    out = _make_gather(idx.shape[0], 56, 2)(p2e, idx)
    return out.reshape(shp + (D_MODEL,))
